# baseline (speedup 1.0000x reference)
import functools

import jax
import jax.numpy as jnp
from jax import lax
from jax.experimental import pallas as pl
from jax.experimental.pallas import tpu as pltpu
from jax.experimental.pallas import tpu_sc as plsc

N = 10000
NP = 10240
E = 160000
EP = 163840
NW = 32
NB = EP // NW // 128
CHK = 2048
NEG = -3.0e38

_MESH = plsc.VectorSubcoreMesh(core_axis_name="c", subcore_axis_name="s")
_DOT00 = (((0,), (0,)), ((), ()))
_DOT01 = (((0,), (1,)), ((), ()))


def _make_gather(D):
    @functools.partial(
        pl.kernel,
        out_type=jax.ShapeDtypeStruct((EP, D), jnp.float32),
        mesh=_MESH,
        scratch_types=[
            pltpu.VMEM((NB, 128), jnp.int32),
            pltpu.VMEM((128, D), jnp.float32),
            pltpu.VMEM_SHARED((NP, D), jnp.float32),
            pltpu.SemaphoreType.DMA,
        ],
    )
    def g(tab_hbm, idx_hbm, out_hbm, idxv, rowv, sp_tab, sem):
        s = lax.axis_index("s")
        w = lax.axis_index("c") * 16 + s
        rows = NP // 16
        pltpu.sync_copy(tab_hbm.at[pl.ds(s * rows, rows)],
                        sp_tab.at[pl.ds(s * rows, rows)])
        pltpu.sync_copy(idx_hbm.at[w], idxv)
        plsc.subcore_barrier()

        def body(j, carry):
            pltpu.async_copy(sp_tab.at[idxv.at[j]], rowv, sem).wait()
            pltpu.sync_copy(rowv, out_hbm.at[pl.ds(w * NB * 128 + j * 128, 128)])
            return carry

        lax.fori_loop(0, NB, body, 0)

    return g


def _make_scatter_add(D):
    @functools.partial(
        pl.kernel,
        out_type=jax.ShapeDtypeStruct((2, NP, D), jnp.float32),
        mesh=_MESH,
        scratch_types=[
            pltpu.VMEM((NB, 128), jnp.int32),
            pltpu.VMEM((128, D), jnp.float32),
            pltpu.VMEM_SHARED((NP, D), jnp.float32),
        ],
    )
    def g(val_hbm, idx_hbm, zero_hbm, out_hbm, idxv, valv, table):
        c = lax.axis_index("c")
        s = lax.axis_index("s")
        w = c * 16 + s
        rows = NP // 16
        pltpu.sync_copy(zero_hbm.at[pl.ds(s * rows, rows)],
                        table.at[pl.ds(s * rows, rows)])
        plsc.subcore_barrier()
        pltpu.sync_copy(idx_hbm.at[w], idxv)

        def body(j, carry):
            pltpu.sync_copy(val_hbm.at[w, j], valv)
            pltpu.sync_copy(valv, table.at[idxv.at[j]], add=True)
            return carry

        lax.fori_loop(0, NB, body, 0)
        plsc.subcore_barrier()
        pltpu.sync_copy(table.at[pl.ds(s * rows, rows)],
                        out_hbm.at[c, pl.ds(s * rows, rows)])

    return g


def _make_gather_scatter_add(D):
    @functools.partial(
        pl.kernel,
        out_type=jax.ShapeDtypeStruct((2, NP, D), jnp.float32),
        mesh=_MESH,
        scratch_types=[
            pltpu.VMEM((NB, 128), jnp.int32),
            pltpu.VMEM((NB, 128), jnp.int32),
            pltpu.VMEM((128, D), jnp.float32),
            pltpu.VMEM_SHARED((NP, D), jnp.float32),
            pltpu.VMEM_SHARED((NP, D), jnp.float32),
            pltpu.SemaphoreType.DMA,
        ],
    )
    def g(tab_hbm, src_hbm, dst_hbm, zero_hbm, out_hbm, idxs, idxd, rowv,
          table, sp_tab, sem):
        c = lax.axis_index("c")
        s = lax.axis_index("s")
        w = c * 16 + s
        rows = NP // 16
        pltpu.sync_copy(zero_hbm.at[pl.ds(s * rows, rows)],
                        table.at[pl.ds(s * rows, rows)])
        pltpu.sync_copy(tab_hbm.at[pl.ds(s * rows, rows)],
                        sp_tab.at[pl.ds(s * rows, rows)])
        plsc.subcore_barrier()
        pltpu.sync_copy(src_hbm.at[w], idxs)
        pltpu.sync_copy(dst_hbm.at[w], idxd)

        def body(j, carry):
            pltpu.async_copy(sp_tab.at[idxs.at[j]], rowv, sem).wait()
            pltpu.sync_copy(rowv, table.at[idxd.at[j]], add=True)
            return carry

        lax.fori_loop(0, NB, body, 0)
        plsc.subcore_barrier()
        pltpu.sync_copy(table.at[pl.ds(s * rows, rows)],
                        out_hbm.at[c, pl.ds(s * rows, rows)])

    return g


def _make_scatter_max():
    nq = EP // 4
    nchunk = nq // CHK

    @functools.partial(
        pl.kernel,
        out_type=jax.ShapeDtypeStruct((4, 64, NP), jnp.float32),
        mesh=_MESH,
        scratch_types=[
            pltpu.VMEM((8, CHK), jnp.float32),
            pltpu.VMEM((CHK,), jnp.int32),
            pltpu.VMEM((8, NP), jnp.float32),
        ],
        compiler_params=pltpu.CompilerParams(needs_layout_passes=False),
    )
    def g(mT_hbm, dst_hbm, neg_hbm, out_hbm, mbuf, dbuf, table):
        c = lax.axis_index("c")
        s = lax.axis_index("s")
        cg = s & 7
        q = c * 2 + (s >> 3)
        lane = lax.broadcasted_iota(jnp.int32, (16,), 0)
        half = (lane >= 8).astype(jnp.int32)
        lane7 = lane & 7

        pltpu.sync_copy(neg_hbm, table)

        def chunk(t, carry):
            off = q * nq + t * CHK
            pltpu.sync_copy(mT_hbm.at[pl.ds(cg * 8, 8), pl.ds(off, CHK)], mbuf)
            pltpu.sync_copy(dst_hbm.at[pl.ds(off, CHK)], dbuf)

            def pair(e2, carry2):
                e = e2 * 2
                ev = jnp.full((16,), e, jnp.int32)
                ia = ev + half
                ib = ev + (1 - half)
                dstv = plsc.load_gather(dbuf, [ia])
                dsw = plsc.load_gather(dbuf, [ib])
                v = plsc.load_gather(mbuf, [lane7, ia])
                vsw = plsc.load_gather(mbuf, [lane7, ib])
                v = jnp.where(dstv == dsw, jnp.maximum(v, vsw), v)
                t0 = plsc.load_gather(table, [lane7, dstv])
                plsc.store_scatter(table, [lane7, dstv], jnp.maximum(t0, v))
                return carry2

            lax.fori_loop(0, CHK // 2, pair, 0)
            return carry

        lax.fori_loop(0, nchunk, chunk, 0)
        pltpu.sync_copy(table, out_hbm.at[q, pl.ds(cg * 8, 8)])

    return g


def _t1_body(efT_ref, xs_ref, W1_ref, b1_ref, W2_ref, b2_ref, I512_ref,
             I32_ref, out_ref):
    hT = jnp.maximum(
        lax.dot_general(W1_ref[...], efT_ref[...], _DOT00,
                        preferred_element_type=jnp.float32) + b1_ref[...], 0.0)
    WeT = lax.dot_general(W2_ref[...], hT, _DOT00,
                          preferred_element_type=jnp.float32) + b2_ref[...]
    xsT = lax.dot_general(xs_ref[...], I512_ref[...], _DOT00,
                          preferred_element_type=jnp.float32)
    acc = WeT[0:32, :] * xsT[0:1, :]
    for i in range(1, 17):
        acc = acc + WeT[32 * i:32 * i + 32, :] * xsT[i:i + 1, :]
    out_ref[...] = lax.dot_general(acc, I32_ref[...], _DOT00,
                                   preferred_element_type=jnp.float32)


def _t2_body(s1_ref, x_ref, Wr_ref, br_ref, WA_ref, bA_ref, WB_ref,
             P_ref, Q_ref):
    x1 = jnp.maximum(
        s1_ref[0] + s1_ref[1]
        + jnp.dot(x_ref[...], Wr_ref[...], preferred_element_type=jnp.float32)
        + br_ref[...], 0.0)
    P_ref[...] = jnp.dot(x1, WA_ref[...],
                         preferred_element_type=jnp.float32) + bA_ref[...]
    Q_ref[...] = jnp.dot(x1, WB_ref[...], preferred_element_type=jnp.float32)


def _t3_body(uP_ref, uQ_ref, Wb_ref, bb_ref, out_ref):
    r = jnp.maximum(uP_ref[...] + uQ_ref[...], 0.0)
    out_ref[...] = lax.dot_general(Wb_ref[...], r, _DOT01,
                                   preferred_element_type=jnp.float32) + bb_ref[...]


def _t4_body(agg_ref, WA_ref, bA_ref, WB_ref, P_ref, Q_ref):
    xm = jnp.maximum(jnp.maximum(agg_ref[0], agg_ref[1]),
                     jnp.maximum(agg_ref[2], agg_ref[3]))
    x2T = jnp.maximum(xm, 0.0)
    P_ref[...] = lax.dot_general(x2T, WA_ref[...], _DOT00,
                                 preferred_element_type=jnp.float32) + bA_ref[...]
    Q_ref[...] = lax.dot_general(x2T, WB_ref[...], _DOT00,
                                 preferred_element_type=jnp.float32)


def _celu(v):
    return jnp.where(v > 0, v, jnp.exp(jnp.minimum(v, 0.0)) - 1.0)


def _t7_body(s4_ref, r4_ref, brel_ref, L1w_ref, L1b_ref, L2w_ref, L2b_ref,
             L3w_ref, L3b_ref, out_ref, acc_ref):
    i = pl.program_id(0)
    x4 = jnp.maximum(s4_ref[0] + s4_ref[1] + r4_ref[...] + brel_ref[...], 0.0)
    rowid = i * 256 + lax.broadcasted_iota(jnp.int32, (256, 1), 0)
    x4 = jnp.where(rowid < N, x4, 0.0)
    bs = jnp.sum(x4, axis=0, keepdims=True)

    @pl.when(i == 0)
    def _():
        acc_ref[...] = bs

    @pl.when(i > 0)
    def _():
        acc_ref[...] = acc_ref[...] + bs

    @pl.when(i == NP // 256 - 1)
    def _():
        pooled = acc_ref[...] * (1.0 / N)
        h1 = _celu(jnp.dot(pooled, L1w_ref[...],
                           preferred_element_type=jnp.float32) + L1b_ref[...])
        h2 = _celu(jnp.dot(h1, L2w_ref[...],
                           preferred_element_type=jnp.float32) + L2b_ref[...])
        out_ref[...] = jnp.dot(h2, L3w_ref[...],
                               preferred_element_type=jnp.float32) + L3b_ref[...]


def _full(shape):
    return pl.BlockSpec(shape, lambda i: tuple(0 for _ in shape))


_gather64 = _make_gather(64)
_gather32 = _make_gather(32)
_scatter_add32 = _make_scatter_add(32)
_scatter_add64 = _make_scatter_add(64)
_scatter_max = _make_scatter_max()


def kernel(x, edge_index, edge_features, W1, b1, W2, b2, Wroot1, broot1, Wa,
           ba, Wb, bb, Wc, bc, Wd, bd, Wrel, brel, Wroot4, L1w, L1b, L2w,
           L2b, L3w, L3b):
    f32 = jnp.float32
    src = edge_index[0]
    dst = edge_index[1]
    padi = (N + jnp.arange(EP - E, dtype=jnp.int32) % 240).astype(jnp.int32)
    srcp = jnp.concatenate([src, padi])
    dstp = jnp.concatenate([dst, padi])
    src_r = srcp.reshape(NW, NB, 128)
    dst_r = dstp.reshape(NW, NB, 128)

    xpad = jnp.pad(x, ((0, NP - N), (0, 32 - 17)))
    efT = jnp.pad(edge_features.T, ((0, 0), (0, EP - E)))

    zeros64 = jnp.zeros((NP, 64), f32)
    zeros32 = jnp.zeros((NP, 32), f32)
    neg8 = jnp.full((8, NP), NEG, f32)
    I512 = jnp.eye(512, dtype=f32)
    I32 = jnp.eye(32, dtype=f32)

    Wroot1p = jnp.zeros((32, 32), f32).at[:17, :].set(Wroot1)
    WaP = Wa[:32] - Wa[32:]
    Wa2 = Wa[32:]
    WcP = Wc[:64] - Wc[64:]
    Wc2 = Wc[64:]
    z64row = jnp.zeros((1, 64), f32)

    xs = jnp.take(xpad, srcp, axis=0)
    nt = EP // 512
    msg = pl.pallas_call(
        _t1_body,
        grid=(nt,),
        in_specs=[
            pl.BlockSpec((4, 512), lambda i: (0, i)),
            pl.BlockSpec((512, 32), lambda i: (i, 0)),
            _full((4, 200)), _full((200, 1)), _full((200, 544)),
            _full((544, 1)), _full((512, 512)), _full((32, 32)),
        ],
        out_specs=pl.BlockSpec((512, 32), lambda i: (i, 0)),
        out_shape=jax.ShapeDtypeStruct((EP, 32), f32),
    )(efT, xs, W1, b1.reshape(200, 1), W2, b2.reshape(544, 1), I512, I32)

    s1 = jnp.stack([jax.ops.segment_sum(msg, dstp, num_segments=NP),
                    zeros32])

    nn = NP // 256
    P2, Q2 = pl.pallas_call(
        _t2_body,
        grid=(nn,),
        in_specs=[
            pl.BlockSpec((2, 256, 32), lambda i: (0, i, 0)),
            pl.BlockSpec((256, 32), lambda i: (i, 0)),
            _full((32, 32)), _full((1, 32)), _full((32, 64)),
            _full((1, 64)), _full((32, 64)),
        ],
        out_specs=[pl.BlockSpec((256, 64), lambda i: (i, 0)),
                   pl.BlockSpec((256, 64), lambda i: (i, 0))],
        out_shape=[jax.ShapeDtypeStruct((NP, 64), f32),
                   jax.ShapeDtypeStruct((NP, 64), f32)],
    )(s1, xpad, Wroot1p, broot1.reshape(1, 32), WaP, ba.reshape(1, 64), Wa2)

    def edge_conv(P, Q, Wlin, blin, WAn, bAn, WBn):
        uP = jnp.take(P, dstp, axis=0)
        uQ = jnp.take(Q, srcp, axis=0)
        mT = pl.pallas_call(
            _t3_body,
            grid=(nt,),
            in_specs=[
                pl.BlockSpec((512, 64), lambda i: (i, 0)),
                pl.BlockSpec((512, 64), lambda i: (i, 0)),
                _full((64, 64)), _full((64, 1)),
            ],
            out_specs=pl.BlockSpec((64, 512), lambda i: (0, i)),
            out_shape=jax.ShapeDtypeStruct((64, EP), f32),
        )(uP, uQ, Wlin, blin.reshape(64, 1))
        aggx = jax.ops.segment_max(mT.T, dstp, num_segments=NP)
        aggx = jnp.where(jnp.isneginf(aggx), NEG, aggx)
        agg = jnp.stack([aggx.T, jnp.full((64, NP), NEG, jnp.float32),
                         jnp.full((64, NP), NEG, jnp.float32),
                         jnp.full((64, NP), NEG, jnp.float32)])
        return pl.pallas_call(
            _t4_body,
            grid=(nn,),
            in_specs=[
                pl.BlockSpec((4, 64, 256), lambda i: (0, 0, i)),
                _full((64, 64)), _full((1, 64)), _full((64, 64)),
            ],
            out_specs=[pl.BlockSpec((256, 64), lambda i: (i, 0)),
                       pl.BlockSpec((256, 64), lambda i: (i, 0))],
            out_shape=[jax.ShapeDtypeStruct((NP, 64), f32),
                       jax.ShapeDtypeStruct((NP, 64), f32)],
        )(agg, WAn, bAn, WBn)

    P3, Q3 = edge_conv(P2, Q2, Wb, bb, WcP, bc.reshape(1, 64), Wc2)
    y, r4 = edge_conv(P3, Q3, Wd, bd, Wrel, z64row, Wroot4)

    ys = jnp.take(y, srcp, axis=0)
    s4 = jnp.stack([jax.ops.segment_sum(ys, dstp, num_segments=NP),
                    zeros64])

    out = pl.pallas_call(
        _t7_body,
        grid=(nn,),
        in_specs=[
            pl.BlockSpec((2, 256, 64), lambda i: (0, i, 0)),
            pl.BlockSpec((256, 64), lambda i: (i, 0)),
            _full((1, 64)), _full((64, 1000)), _full((1, 1000)),
            _full((1000, 1000)), _full((1, 1000)), _full((1000, 1)),
            _full((1, 1)),
        ],
        out_specs=_full((1, 1)),
        out_shape=jax.ShapeDtypeStruct((1, 1), f32),
        scratch_shapes=[pltpu.VMEM((1, 64), f32)],
    )(s4, r4, brel.reshape(1, 64), L1w, L1b.reshape(1, 1000), L2w,
      L2b.reshape(1, 1000), L3w, L3b.reshape(1, 1))
    return out
